# baseline (speedup 1.0000x reference)
import jax
import jax.numpy as jnp
from jax.experimental import pallas as pl
from jax.experimental.pallas import tpu as pltpu

_N_ROWS = 128
_N_COLS = 100000
_W = 4096
_ROW_BLK = 128
_RS = _ROW_BLK // 8
_NB = -(-_N_COLS // _W)

_R1 = (13, 15, 26, 6)
_R2 = (17, 29, 16, 24)


def _rotl(x, r):
    return (x << jnp.uint32(r)) | (x >> jnp.uint32(32 - r))


def _four_rounds(x0, x1, rots):
    for r in rots:
        x0 = x0 + x1
        x1 = _rotl(x1, r) ^ x0
    return x0, x1


def _gumbel_bits(n42):
    ks1 = jnp.uint32(42)
    ks2 = jnp.uint32(0 ^ 42 ^ 0x1BD11BDA)
    x0 = jnp.zeros_like(n42)
    x1 = n42
    x0, x1 = _four_rounds(x0, x1, _R1)
    x0, x1 = x0 + ks1, x1 + (ks2 + jnp.uint32(1))
    x0, x1 = _four_rounds(x0, x1, _R2)
    x0, x1 = x0 + ks2, x1 + jnp.uint32(2)
    x0, x1 = _four_rounds(x0, x1, _R1)
    x0, x1 = x0, x1 + (ks1 + jnp.uint32(3))
    x0, x1 = _four_rounds(x0, x1, _R2)
    x0, x1 = x0 + ks1, x1 + (ks2 + jnp.uint32(4))
    x0, x1 = _four_rounds(x0, x1, _R1)
    x0, x1 = x0 + ks2, x1 + jnp.uint32(5)
    return x0 ^ x1


def _gumbel(bits):
    fb = (bits >> jnp.uint32(9)) | jnp.uint32(0x3F800000)
    floats = jax.lax.bitcast_convert_type(fb, jnp.float32) - jnp.float32(1.0)
    tiny = jnp.float32(jnp.finfo(jnp.float32).tiny)
    u = jnp.maximum(tiny, floats * (jnp.float32(1.0) - tiny) + tiny)
    return -jnp.log(-jnp.log(u))


def _sampler_kernel(x_ref, out_ref, m_ref, i_ref):
    r = pl.program_id(0)
    b = pl.program_id(1)

    @pl.when(b == 0)
    def _init():
        m_ref[...] = jnp.full((_ROW_BLK, 1), -jnp.inf, jnp.float32)
        i_ref[...] = jnp.zeros((_ROW_BLK, 1), jnp.int32)

    lane = jax.lax.broadcasted_iota(jnp.int32, (8, _W), 1)
    rowi = jax.lax.broadcasted_iota(jnp.int32, (8, _W), 0)
    colg = b * _W + lane

    for rs in range(_RS):
        row = r * _ROW_BLK + rs * 8 + rowi
        n42 = (row * _N_COLS + colg + 42).astype(jnp.uint32)
        g = _gumbel(_gumbel_bits(n42))
        x = x_ref[rs * 8:(rs + 1) * 8, :]
        phi = jnp.where(colg < _N_COLS, x + g, -jnp.inf)

        m = jnp.max(phi, axis=1, keepdims=True)
        idx = jnp.min(
            jnp.where(phi == m, colg, jnp.int32(2**30)),
            axis=1, keepdims=True,
        )
        mm = m_ref[rs * 8:(rs + 1) * 8, :]
        better = m > mm
        i_ref[rs * 8:(rs + 1) * 8, :] = jnp.where(
            better, idx, i_ref[rs * 8:(rs + 1) * 8, :]
        )
        m_ref[rs * 8:(rs + 1) * 8, :] = jnp.where(better, m, mm)

    @pl.when(b == _NB - 1)
    def _done():
        out_ref[...] = i_ref[...]


def kernel(logits):
    out = pl.pallas_call(
        _sampler_kernel,
        grid=(_N_ROWS // _ROW_BLK, _NB),
        in_specs=[
            pl.BlockSpec((_ROW_BLK, _W), lambda r, b: (r, b)),
        ],
        out_specs=pl.BlockSpec((_ROW_BLK, 1), lambda r, b: (r, 0)),
        out_shape=jax.ShapeDtypeStruct((_N_ROWS, 1), jnp.int32),
        scratch_shapes=[
            pltpu.VMEM((_ROW_BLK, 1), jnp.float32),
            pltpu.VMEM((_ROW_BLK, 1), jnp.int32),
        ],
        compiler_params=pltpu.CompilerParams(
            dimension_semantics=("arbitrary", "arbitrary"),
        ),
    )(logits)
    return out.reshape(_N_ROWS)


# speedup vs baseline: 1.0912x; 1.0912x over previous
import jax
import jax.numpy as jnp
from jax.experimental import pallas as pl
from jax.experimental.pallas import tpu as pltpu

_N_ROWS = 128
_N_COLS = 100000
_W = 2048
_ROW_BLK = 128
_RS = _ROW_BLK // 8
_NB = -(-_N_COLS // _W)

_R1 = (13, 15, 26, 6)
_R2 = (17, 29, 16, 24)


def _rotl(x, r):
    return (x << jnp.uint32(r)) | (x >> jnp.uint32(32 - r))


def _four_rounds(x0, x1, rots):
    for r in rots:
        x0 = x0 + x1
        x1 = _rotl(x1, r) ^ x0
    return x0, x1


def _gumbel_bits(n42):
    ks1 = jnp.uint32(42)
    ks2 = jnp.uint32(0 ^ 42 ^ 0x1BD11BDA)
    x0 = jnp.zeros_like(n42)
    x1 = n42
    x0, x1 = _four_rounds(x0, x1, _R1)
    x0, x1 = x0 + ks1, x1 + (ks2 + jnp.uint32(1))
    x0, x1 = _four_rounds(x0, x1, _R2)
    x0, x1 = x0 + ks2, x1 + jnp.uint32(2)
    x0, x1 = _four_rounds(x0, x1, _R1)
    x0, x1 = x0, x1 + (ks1 + jnp.uint32(3))
    x0, x1 = _four_rounds(x0, x1, _R2)
    x0, x1 = x0 + ks1, x1 + (ks2 + jnp.uint32(4))
    x0, x1 = _four_rounds(x0, x1, _R1)
    x0, x1 = x0 + ks2, x1 + jnp.uint32(5)
    return x0 ^ x1


def _gumbel(bits):
    fb = (bits >> jnp.uint32(9)) | jnp.uint32(0x3F800000)
    floats = jax.lax.bitcast_convert_type(fb, jnp.float32) - jnp.float32(1.0)
    tiny = jnp.float32(jnp.finfo(jnp.float32).tiny)
    u = jnp.maximum(tiny, floats * (jnp.float32(1.0) - tiny) + tiny)
    return -jnp.log(-jnp.log(u))


def _sampler_kernel(x_ref, out_ref, m_ref, i_ref):
    r = pl.program_id(0)
    b = pl.program_id(1)

    @pl.when(b == 0)
    def _init():
        m_ref[...] = jnp.full((_ROW_BLK, 1), -jnp.inf, jnp.float32)
        i_ref[...] = jnp.zeros((_ROW_BLK, 1), jnp.int32)

    lane = jax.lax.broadcasted_iota(jnp.int32, (8, _W), 1)
    rowi = jax.lax.broadcasted_iota(jnp.int32, (8, _W), 0)
    colg = b * _W + lane

    for rs in range(_RS):
        row = r * _ROW_BLK + rs * 8 + rowi
        n42 = (row * _N_COLS + colg + 42).astype(jnp.uint32)
        g = _gumbel(_gumbel_bits(n42))
        x = x_ref[rs * 8:(rs + 1) * 8, :]
        phi = jnp.where(colg < _N_COLS, x + g, -jnp.inf)

        m = jnp.max(phi, axis=1, keepdims=True)
        idx = jnp.min(
            jnp.where(phi == m, colg, jnp.int32(2**30)),
            axis=1, keepdims=True,
        )
        mm = m_ref[rs * 8:(rs + 1) * 8, :]
        better = m > mm
        i_ref[rs * 8:(rs + 1) * 8, :] = jnp.where(
            better, idx, i_ref[rs * 8:(rs + 1) * 8, :]
        )
        m_ref[rs * 8:(rs + 1) * 8, :] = jnp.where(better, m, mm)

    @pl.when(b == _NB - 1)
    def _done():
        out_ref[...] = i_ref[...]


def kernel(logits):
    out = pl.pallas_call(
        _sampler_kernel,
        grid=(_N_ROWS // _ROW_BLK, _NB),
        in_specs=[
            pl.BlockSpec((_ROW_BLK, _W), lambda r, b: (r, b)),
        ],
        out_specs=pl.BlockSpec((_ROW_BLK, 1), lambda r, b: (r, 0)),
        out_shape=jax.ShapeDtypeStruct((_N_ROWS, 1), jnp.int32),
        scratch_shapes=[
            pltpu.VMEM((_ROW_BLK, 1), jnp.float32),
            pltpu.VMEM((_ROW_BLK, 1), jnp.int32),
        ],
        compiler_params=pltpu.CompilerParams(
            dimension_semantics=("arbitrary", "arbitrary"),
        ),
    )(logits)
    return out.reshape(_N_ROWS)


# speedup vs baseline: 1.1715x; 1.0736x over previous
import jax
import jax.numpy as jnp
from jax.experimental import pallas as pl
from jax.experimental.pallas import tpu as pltpu

_N_ROWS = 128
_N_COLS = 100000
_W = 2048
_ROW_BLK = 128
_RS = _ROW_BLK // 8
_NB = -(-_N_COLS // _W)

_R1 = (13, 15, 26, 6)
_R2 = (17, 29, 16, 24)


def _rotl(x, r):
    return (x << jnp.uint32(r)) | (x >> jnp.uint32(32 - r))


def _four_rounds(x0, x1, rots):
    for r in rots:
        x0 = x0 + x1
        x1 = _rotl(x1, r) ^ x0
    return x0, x1


def _gumbel_bits(n42):
    ks1 = jnp.uint32(42)
    ks2 = jnp.uint32(0 ^ 42 ^ 0x1BD11BDA)
    x0 = jnp.zeros_like(n42)
    x1 = n42
    x0, x1 = _four_rounds(x0, x1, _R1)
    x0, x1 = x0 + ks1, x1 + (ks2 + jnp.uint32(1))
    x0, x1 = _four_rounds(x0, x1, _R2)
    x0, x1 = x0 + ks2, x1 + jnp.uint32(2)
    x0, x1 = _four_rounds(x0, x1, _R1)
    x0, x1 = x0, x1 + (ks1 + jnp.uint32(3))
    x0, x1 = _four_rounds(x0, x1, _R2)
    x0, x1 = x0 + ks1, x1 + (ks2 + jnp.uint32(4))
    x0, x1 = _four_rounds(x0, x1, _R1)
    x0, x1 = x0 + ks2, x1 + jnp.uint32(5)
    return x0 ^ x1


def _gumbel(bits):
    fb = (bits >> jnp.uint32(9)) | jnp.uint32(0x3F800000)
    floats = jax.lax.bitcast_convert_type(fb, jnp.float32) - jnp.float32(1.0)
    tiny = jnp.float32(jnp.finfo(jnp.float32).tiny)
    u = jnp.maximum(tiny, floats * (jnp.float32(1.0) - tiny) + tiny)
    return -jnp.log(-jnp.log(u))


_TAIL = (_N_COLS - (_NB - 1) * _W) // 128 * 128


def _sampler_kernel(x_ref, out_ref, bv_ref, bt_ref):
    b = pl.program_id(0)

    @pl.when(b == 0)
    def _init():
        bv_ref[...] = jnp.full((_N_ROWS, _W), -jnp.inf, jnp.float32)
        bt_ref[...] = jnp.zeros((_N_ROWS, _W), jnp.int32)

    lane = jax.lax.broadcasted_iota(jnp.int32, (8, _W), 1)
    rowi = jax.lax.broadcasted_iota(jnp.int32, (8, _W), 0)
    colg = b * _W + lane

    for rs in range(_RS):
        row = rs * 8 + rowi
        n42 = (row * _N_COLS + colg + 42).astype(jnp.uint32)
        g = _gumbel(_gumbel_bits(n42))
        x = x_ref[rs * 8:(rs + 1) * 8, :]
        phi = x + g

        r0, r1 = rs * 8, (rs + 1) * 8
        bv = bv_ref[r0:r1, :_TAIL]
        upd = phi[:, :_TAIL] > bv
        bv_ref[r0:r1, :_TAIL] = jnp.where(upd, phi[:, :_TAIL], bv)
        bt_ref[r0:r1, :_TAIL] = jnp.where(upd, b, bt_ref[r0:r1, :_TAIL])

        phi_t = jnp.where(colg[:, _TAIL:] < _N_COLS, phi[:, _TAIL:], -jnp.inf)
        bvt = bv_ref[r0:r1, _TAIL:]
        updt = phi_t > bvt
        bv_ref[r0:r1, _TAIL:] = jnp.where(updt, phi_t, bvt)
        bt_ref[r0:r1, _TAIL:] = jnp.where(updt, b, bt_ref[r0:r1, _TAIL:])

    @pl.when(b == _NB - 1)
    def _done():
        for rs in range(_RS):
            r0, r1 = rs * 8, (rs + 1) * 8
            bv = bv_ref[r0:r1, :]
            m = jnp.max(bv, axis=1, keepdims=True)
            colw = bt_ref[r0:r1, :] * _W + lane
            idx = jnp.min(
                jnp.where(bv == m, colw, jnp.int32(2**30)),
                axis=1, keepdims=True,
            )
            out_ref[r0:r1, :] = idx


def kernel(logits):
    out = pl.pallas_call(
        _sampler_kernel,
        grid=(_NB,),
        in_specs=[
            pl.BlockSpec((_ROW_BLK, _W), lambda b: (0, b)),
        ],
        out_specs=pl.BlockSpec((_ROW_BLK, 1), lambda b: (0, 0)),
        out_shape=jax.ShapeDtypeStruct((_N_ROWS, 1), jnp.int32),
        scratch_shapes=[
            pltpu.VMEM((_N_ROWS, _W), jnp.float32),
            pltpu.VMEM((_N_ROWS, _W), jnp.int32),
        ],
        compiler_params=pltpu.CompilerParams(
            dimension_semantics=("arbitrary",),
        ),
    )(logits)
    return out.reshape(_N_ROWS)


# speedup vs baseline: 1.1799x; 1.0071x over previous
import jax
import jax.numpy as jnp
from jax.experimental import pallas as pl
from jax.experimental.pallas import tpu as pltpu

_N_ROWS = 128
_N_COLS = 100000
_W = 2048
_ROW_BLK = 128
_RS = _ROW_BLK // 8
_NB = -(-_N_COLS // _W)

_R1 = (13, 15, 26, 6)
_R2 = (17, 29, 16, 24)


def _rotl(x, r):
    return (x << jnp.uint32(r)) | (x >> jnp.uint32(32 - r))


def _four_rounds(x0, x1, rots):
    for r in rots:
        x0 = x0 + x1
        x1 = _rotl(x1, r) ^ x0
    return x0, x1


def _gumbel_bits(n42):
    ks1 = jnp.uint32(42)
    ks2 = jnp.uint32(0 ^ 42 ^ 0x1BD11BDA)
    x0 = jnp.zeros_like(n42)
    x1 = n42
    x0, x1 = _four_rounds(x0, x1, _R1)
    x0, x1 = x0 + ks1, x1 + (ks2 + jnp.uint32(1))
    x0, x1 = _four_rounds(x0, x1, _R2)
    x0, x1 = x0 + ks2, x1 + jnp.uint32(2)
    x0, x1 = _four_rounds(x0, x1, _R1)
    x0, x1 = x0, x1 + (ks1 + jnp.uint32(3))
    x0, x1 = _four_rounds(x0, x1, _R2)
    x0, x1 = x0 + ks1, x1 + (ks2 + jnp.uint32(4))
    x0, x1 = _four_rounds(x0, x1, _R1)
    x0, x1 = x0 + ks2, x1 + jnp.uint32(5)
    return x0 ^ x1


def _gumbel(bits):
    fb = (bits >> jnp.uint32(9)) | jnp.uint32(0x3F800000)
    floats = jax.lax.bitcast_convert_type(fb, jnp.float32) - jnp.float32(1.0)
    tiny = jnp.float32(jnp.finfo(jnp.float32).tiny)
    u = floats + tiny
    return -jnp.log(-jnp.log(u))


_TAIL = (_N_COLS - (_NB - 1) * _W) // 128 * 128


def _sampler_kernel(x_ref, out_ref, bv_ref, bt_ref):
    b = pl.program_id(0)

    @pl.when(b == 0)
    def _init():
        bv_ref[...] = jnp.full((_N_ROWS, _W), -jnp.inf, jnp.float32)
        bt_ref[...] = jnp.zeros((_N_ROWS, _W), jnp.int32)

    lane = jax.lax.broadcasted_iota(jnp.int32, (8, _W), 1)
    rowi = jax.lax.broadcasted_iota(jnp.int32, (8, _W), 0)
    tail_bound = _N_COLS - b * _W

    for rs in range(_RS):
        row = rs * 8 + rowi
        n42 = (row * _N_COLS + lane + (b * _W + 42)).astype(jnp.uint32)
        g = _gumbel(_gumbel_bits(n42))
        x = x_ref[rs * 8:(rs + 1) * 8, :]
        phi = x + g

        r0, r1 = rs * 8, (rs + 1) * 8
        bv = bv_ref[r0:r1, :_TAIL]
        upd = phi[:, :_TAIL] > bv
        bv_ref[r0:r1, :_TAIL] = jnp.where(upd, phi[:, :_TAIL], bv)
        bt_ref[r0:r1, :_TAIL] = jnp.where(upd, b, bt_ref[r0:r1, :_TAIL])

        phi_t = jnp.where(lane[:, _TAIL:] < tail_bound, phi[:, _TAIL:], -jnp.inf)
        bvt = bv_ref[r0:r1, _TAIL:]
        updt = phi_t > bvt
        bv_ref[r0:r1, _TAIL:] = jnp.where(updt, phi_t, bvt)
        bt_ref[r0:r1, _TAIL:] = jnp.where(updt, b, bt_ref[r0:r1, _TAIL:])

    @pl.when(b == _NB - 1)
    def _done():
        for rs in range(_RS):
            r0, r1 = rs * 8, (rs + 1) * 8
            bv = bv_ref[r0:r1, :]
            m = jnp.max(bv, axis=1, keepdims=True)
            colw = bt_ref[r0:r1, :] * _W + lane
            idx = jnp.min(
                jnp.where(bv == m, colw, jnp.int32(2**30)),
                axis=1, keepdims=True,
            )
            out_ref[r0:r1, :] = idx


def kernel(logits):
    out = pl.pallas_call(
        _sampler_kernel,
        grid=(_NB,),
        in_specs=[
            pl.BlockSpec((_ROW_BLK, _W), lambda b: (0, b)),
        ],
        out_specs=pl.BlockSpec((_ROW_BLK, 1), lambda b: (0, 0)),
        out_shape=jax.ShapeDtypeStruct((_N_ROWS, 1), jnp.int32),
        scratch_shapes=[
            pltpu.VMEM((_N_ROWS, _W), jnp.float32),
            pltpu.VMEM((_N_ROWS, _W), jnp.int32),
        ],
        compiler_params=pltpu.CompilerParams(
            dimension_semantics=("arbitrary",),
        ),
    )(logits)
    return out.reshape(_N_ROWS)
